# Initial kernel scaffold; baseline (speedup 1.0000x reference)
import jax
import jax.numpy as jnp
from jax.experimental import pallas as pl


def kernel(x, snp_gene, W, bias):
    raise NotImplementedError("write your pallas kernel here")


# speedup vs baseline: 83.6703x; 83.6703x over previous
import functools

import jax
import jax.numpy as jnp
from jax import lax
from jax.experimental import pallas as pl
from jax.experimental.pallas import tpu as pltpu

_K = 256
_G = 64


def _tile_kernel(jlo_ref, jhi_ref, x3_ref, wt3_ref, g3_ref, bias_ref, out_ref,
                 *, G, K, FG):
    t = pl.program_id(0)
    base = t * G
    out_ref[...] = jnp.zeros_like(out_ref)

    gcol = lax.broadcasted_iota(jnp.int32, (FG, K), 0) // 4

    def body(j, carry):
        xk = x3_ref[j]
        wkT = wt3_ref[j]
        gk = g3_ref[pl.ds(j, 1), :]
        mask = gk == base + gcol
        wsel = jnp.broadcast_to(wkT[None, :, :], (G, 4, K)).reshape(FG, K)
        ft = jnp.where(mask, wsel, 0.0)
        out_ref[...] += lax.dot_general(
            xk, ft, (((1,), (1,)), ((), ())),
            preferred_element_type=jnp.float32)
        return carry

    lax.fori_loop(jlo_ref[t], jhi_ref[t], body, 0)
    out_ref[...] = jnp.tanh(out_ref[...] + bias_ref[0])


def kernel(x, snp_gene, W, bias):
    B, NS = x.shape
    NG, L = bias.shape
    K, G = _K, _G
    FG = L * G

    NSUB = (NS + K - 1) // K
    NT = (NG + G - 1) // G
    NGP = NT * G

    sg = snp_gene.astype(jnp.int32)
    pad = NSUB * K - NS
    if pad:
        x = jnp.pad(x, ((0, 0), (0, pad)))
        sg = jnp.pad(sg, (0, pad), constant_values=NGP)
        W = jnp.pad(W, ((0, pad), (0, 0)))

    x3 = x.reshape(B, NSUB, K).transpose(1, 0, 2)
    wt3 = W.T.reshape(L, NSUB, K).transpose(1, 0, 2)
    g3 = sg.reshape(NSUB, K)
    biasp = jnp.pad(bias, ((0, NGP - NG), (0, 0))).reshape(NT, 1, FG)

    bnd = (jnp.arange(NT + 1, dtype=jnp.int32) * G).astype(sg.dtype)
    starts = jnp.searchsorted(sg, bnd).astype(jnp.int32)
    jlo = starts[:-1] // K
    jhi = (starts[1:] + K - 1) // K

    grid_spec = pltpu.PrefetchScalarGridSpec(
        num_scalar_prefetch=2,
        grid=(NT,),
        in_specs=[
            pl.BlockSpec(x3.shape, lambda t, lo, hi: (0, 0, 0)),
            pl.BlockSpec(wt3.shape, lambda t, lo, hi: (0, 0, 0)),
            pl.BlockSpec(g3.shape, lambda t, lo, hi: (0, 0)),
            pl.BlockSpec((1, 1, FG), lambda t, lo, hi: (t, 0, 0)),
        ],
        out_specs=pl.BlockSpec((B, FG), lambda t, lo, hi: (0, t)),
    )
    out = pl.pallas_call(
        functools.partial(_tile_kernel, G=G, K=K, FG=FG),
        grid_spec=grid_spec,
        out_shape=jax.ShapeDtypeStruct((B, NT * FG), jnp.float32),
    )(jlo, jhi, x3, wt3, g3, biasp)
    return out[:, : NG * L]
